# Initial kernel scaffold; baseline (speedup 1.0000x reference)
import jax
import jax.numpy as jnp
from jax.experimental import pallas as pl


def kernel(x, edge_index, W1, b1, W2, b2):
    raise NotImplementedError("write your pallas kernel here")


# speedup vs baseline: 34.2749x; 34.2749x over previous
import functools

import jax
import jax.numpy as jnp
from jax import lax
from jax.experimental import pallas as pl
from jax.experimental.pallas import tpu as pltpu
from jax.experimental.pallas import tpu_sc as plsc

N = 100_000
E = 6_400_000

PADROWS = 352
NP = N + PADROWS
NR = NP // 128
RPT = NP // 16

W = 128
K = 8
EPW = 50_176
EP = EPW * W
T13 = EPW // 32
G13 = T13 // K
T5 = EPW // 16
G5 = T5 // K

NC, NS = 2, 16

_mesh = plsc.VectorSubcoreMesh(core_axis_name="c", subcore_axis_name="s")


def _fill_zeros_1d(ref, n):
    def body(i, _):
        ref[pl.ds(i * 16, 16)] = jnp.zeros((16,), jnp.float32)
        return 0
    lax.fori_loop(0, n // 16, body, 0)


def _sc_degree(dst2d):
    @functools.partial(
        pl.kernel,
        out_type=(jax.ShapeDtypeStruct((NP,), jnp.float32),
                  jax.ShapeDtypeStruct((NP,), jnp.float32)),
        mesh=_mesh,
        scratch_types=[
            pltpu.VMEM((K, W), jnp.int32),
            pltpu.VMEM((W,), jnp.float32),
            pltpu.VMEM((RPT,), jnp.float32),
            pltpu.VMEM_SHARED((NP,), jnp.float32),
        ],
    )
    def k(dst_hbm, out0_hbm, out1_hbm, didx, ones_v, zb, acc):
        cid = lax.axis_index("c")
        sid = lax.axis_index("s")
        wid = sid * NC + cid
        _fill_zeros_1d(zb, RPT)
        for i in range(W // 16):
            ones_v[pl.ds(i * 16, 16)] = jnp.ones((16,), jnp.float32)
        pltpu.sync_copy(zb, acc.at[pl.ds(sid * RPT, RPT)])
        plsc.subcore_barrier()
        base_w = wid * T13

        def body(g, _):
            w0 = base_w + g * K
            pltpu.sync_copy(dst_hbm.at[pl.ds(w0, K)], didx)
            for j in range(K):
                pltpu.sync_copy(ones_v, acc.at[didx.at[j]], add=True)
            return 0

        lax.fori_loop(0, G13, body, 0)
        plsc.subcore_barrier()

        @pl.when(cid == 0)
        def _():
            pltpu.sync_copy(acc.at[pl.ds(sid * RPT, RPT)],
                            out0_hbm.at[pl.ds(sid * RPT, RPT)])

        @pl.when(cid == 1)
        def _():
            pltpu.sync_copy(acc.at[pl.ds(sid * RPT, RPT)],
                            out1_hbm.at[pl.ds(sid * RPT, RPT)])

    return k(dst2d)


def _sc_z1(y1flat, src2d, dst2d):
    @functools.partial(
        pl.kernel,
        out_type=(jax.ShapeDtypeStruct((NP,), jnp.float32),
                  jax.ShapeDtypeStruct((NP,), jnp.float32)),
        mesh=_mesh,
        scratch_types=[
            pltpu.VMEM((K, W), jnp.int32),
            pltpu.VMEM((K, W), jnp.int32),
            pltpu.VMEM((K, W), jnp.float32),
            pltpu.VMEM((RPT,), jnp.float32),
            pltpu.VMEM_SHARED((NP,), jnp.float32),
            pltpu.VMEM_SHARED((NP,), jnp.float32),
            pltpu.SemaphoreType.DMA,
        ],
    )
    def k(y1_hbm, src_hbm, dst_hbm, out0_hbm, out1_hbm, sidx, didx, gv, stage,
          y1sh, acc, sem):
        cid = lax.axis_index("c")
        sid = lax.axis_index("s")
        wid = sid * NC + cid
        pltpu.sync_copy(y1_hbm.at[pl.ds(sid * RPT, RPT)], stage)
        pltpu.sync_copy(stage, y1sh.at[pl.ds(sid * RPT, RPT)])
        _fill_zeros_1d(stage, RPT)
        pltpu.sync_copy(stage, acc.at[pl.ds(sid * RPT, RPT)])
        plsc.subcore_barrier()
        base_w = wid * T13

        def body(g, _):
            w0 = base_w + g * K
            pltpu.sync_copy(src_hbm.at[pl.ds(w0, K)], sidx)
            pltpu.sync_copy(dst_hbm.at[pl.ds(w0, K)], didx)
            descs = [pltpu.async_copy(y1sh.at[sidx.at[j]], gv.at[j], sem)
                     for j in range(K)]
            for d in descs:
                d.wait()
            for j in range(K):
                pltpu.sync_copy(gv.at[j], acc.at[didx.at[j]], add=True)
            return 0

        lax.fori_loop(0, G13, body, 0)
        plsc.subcore_barrier()

        @pl.when(cid == 0)
        def _():
            pltpu.sync_copy(acc.at[pl.ds(sid * RPT, RPT)],
                            out0_hbm.at[pl.ds(sid * RPT, RPT)])

        @pl.when(cid == 1)
        def _():
            pltpu.sync_copy(acc.at[pl.ds(sid * RPT, RPT)],
                            out1_hbm.at[pl.ds(sid * RPT, RPT)])

    return k(y1flat, src2d, dst2d)


def _sc_z2(src2d, dst2d, t0, t1, t2, t3):
    @functools.partial(
        pl.kernel,
        out_type=jax.ShapeDtypeStruct((4, NP, 16), jnp.float32),
        mesh=_mesh,
        scratch_types=[
            pltpu.VMEM((K, W), jnp.int32),
            pltpu.VMEM((K, W), jnp.int32),
            pltpu.VMEM((K, W, 16), jnp.float32),
            pltpu.VMEM((392, 16), jnp.float32),
            pltpu.VMEM_SHARED((NP, 16), jnp.float32),
            pltpu.SemaphoreType.DMA,
        ],
        compiler_params=pltpu.CompilerParams(use_tc_tiling_on_sc=False),
    )
    def k(src_hbm, dst_hbm, t0h, t1h, t2h, t3h, out_hbm, sidx, didx, vals,
          zb2, acc2, sem):
        cid = lax.axis_index("c")
        sid = lax.axis_index("s")

        def zfill(i, _):
            zb2[i, :] = jnp.zeros((16,), jnp.float32)
            return 0

        lax.fori_loop(0, 392, zfill, 0)
        base_w = sid * T5

        def do_chunk(tab, ck):
            for q in range(16):
                pltpu.sync_copy(zb2, acc2.at[pl.ds(sid * RPT + q * 392, 392)])
            plsc.subcore_barrier()

            def body(g, _):
                w0 = base_w + g * K
                pltpu.sync_copy(src_hbm.at[pl.ds(w0, K)], sidx)
                pltpu.sync_copy(dst_hbm.at[pl.ds(w0, K)], didx)
                descs = [pltpu.async_copy(tab.at[sidx.at[j]], vals.at[j], sem)
                         for j in range(K)]
                for d in descs:
                    d.wait()
                for j in range(K):
                    pltpu.sync_copy(vals.at[j], acc2.at[didx.at[j]], add=True)
                return 0

            lax.fori_loop(0, G5, body, 0)
            plsc.subcore_barrier()
            pltpu.sync_copy(acc2.at[pl.ds(sid * RPT, RPT)],
                            out_hbm.at[ck, pl.ds(sid * RPT, RPT)])
            plsc.subcore_barrier()

        @pl.when(cid == 0)
        def _():
            do_chunk(t0h, 0)
            do_chunk(t1h, 1)

        @pl.when(cid == 1)
        def _():
            do_chunk(t2h, 2)
            do_chunk(t3h, 3)

    return k(src2d, dst2d, t0, t1, t2, t3)


def _tc_prep(deg2g, xg):
    def body(d_ref, x_ref, dinv_ref, y1_ref):
        deg = d_ref[0] + d_ref[1] + 1.0
        dinv = lax.rsqrt(deg)
        dinv_ref[...] = dinv
        y1_ref[...] = dinv * x_ref[...]

    return pl.pallas_call(
        body,
        out_shape=(jax.ShapeDtypeStruct((NR, 128), jnp.float32),
                   jax.ShapeDtypeStruct((NR, 128), jnp.float32)),
    )(deg2g, xg)


def _tc_s1(z1pg, dinvg, y1g):
    def body(z_ref, dinv_ref, y1_ref, s1_ref):
        s1_ref[...] = dinv_ref[...] * (z_ref[0] + z_ref[1] + y1_ref[...])

    return pl.pallas_call(
        body,
        out_shape=jax.ShapeDtypeStruct((NR, 128), jnp.float32),
    )(z1pg, dinvg, y1g)


_BLK2 = 512
_GRID2 = NP // _BLK2


def _tc_layer1(s1c, dinvc, w1, b1):
    def body(s1_ref, dinv_ref, w1_ref, b1_ref, o0, o1, o2, o3):
        i = pl.program_id(0)
        h = jnp.maximum(s1_ref[...] * w1_ref[...] + b1_ref[...], 0.0)
        y2 = dinv_ref[...] * h
        row = i * _BLK2 + lax.broadcasted_iota(jnp.int32, (_BLK2, 1), 0)
        y2 = jnp.where(row < N, y2, 0.0)
        o0[...] = y2[:, 0:16]
        o1[...] = y2[:, 16:32]
        o2[...] = y2[:, 32:48]
        o3[...] = y2[:, 48:64]

    cspec = pl.BlockSpec((_BLK2, 16), lambda i: (i, 0))
    return pl.pallas_call(
        body,
        grid=(_GRID2,),
        in_specs=[
            pl.BlockSpec((_BLK2, 1), lambda i: (i, 0)),
            pl.BlockSpec((_BLK2, 1), lambda i: (i, 0)),
            pl.BlockSpec((1, 64), lambda i: (0, 0)),
            pl.BlockSpec((1, 64), lambda i: (0, 0)),
        ],
        out_specs=[cspec, cspec, cspec, cspec],
        out_shape=tuple(jax.ShapeDtypeStruct((NP, 16), jnp.float32)
                        for _ in range(4)),
    )(s1c, dinvc, w1, b1)


_BLK3 = 800
_GRID3 = N // _BLK3


def _tc_final(y2c, z2c, dinvc, w2, b2):
    def body(y0, y1_, y2_, y3, z0, z1_, z2_, z3, dinv_ref, w2_ref, b2_ref,
             out_ref):
        yy = jnp.concatenate([y0[...], y1_[...], y2_[...], y3[...]], axis=1)
        zz = jnp.concatenate([z0[...], z1_[...], z2_[...], z3[...]], axis=1)
        g = dinv_ref[...] * (zz + yy)
        logits = jnp.dot(g, w2_ref[...], preferred_element_type=jnp.float32)
        logits = logits + b2_ref[...]
        m = jnp.max(logits, axis=1, keepdims=True)
        lse = jnp.log(jnp.sum(jnp.exp(logits - m), axis=1, keepdims=True)) + m
        out_ref[...] = logits - lse

    cspec = pl.BlockSpec((_BLK3, 16), lambda i: (i, 0))
    return pl.pallas_call(
        body,
        grid=(_GRID3,),
        in_specs=[cspec] * 8 + [
            pl.BlockSpec((_BLK3, 1), lambda i: (i, 0)),
            pl.BlockSpec((64, 128), lambda i: (0, 0)),
            pl.BlockSpec((1, 128), lambda i: (0, 0)),
        ],
        out_specs=pl.BlockSpec((_BLK3, 128), lambda i: (i, 0)),
        out_shape=jax.ShapeDtypeStruct((N, 128), jnp.float32),
    )(*y2c, *z2c, dinvc, w2, b2)


def kernel(x, edge_index, W1, b1, W2, b2):
    src = edge_index[0]
    dst = edge_index[1]
    pad = EP - E
    padidx = (N + (jnp.arange(pad, dtype=jnp.int32) % PADROWS)).astype(jnp.int32)
    src2d = jnp.concatenate([src, padidx]).reshape(EPW, W)
    dst2d = jnp.concatenate([dst, padidx]).reshape(EPW, W)

    d0, d1 = _sc_degree(dst2d)
    deg2 = jnp.stack([d0, d1]).reshape(NC, NR, 128)
    xg = jnp.pad(x[:, 0], (0, PADROWS)).reshape(NR, 128)
    dinvg, y1g = _tc_prep(deg2, xg)

    zp0, zp1 = _sc_z1(y1g.reshape(NP), src2d, dst2d)
    z1p = jnp.stack([zp0, zp1]).reshape(NC, NR, 128)
    s1g = _tc_s1(z1p, dinvg, y1g)

    s1c = s1g.reshape(NP, 1)
    dinvc = dinvg.reshape(NP, 1)
    y2c = _tc_layer1(s1c, dinvc, W1.reshape(1, 64), b1.reshape(1, 64))

    z2 = _sc_z2(src2d, dst2d, *y2c)
    z2c = [z2[i] for i in range(4)]
    return _tc_final(y2c, z2c, dinvc, W2, b2.reshape(1, 128))
